# baseline (speedup 1.0000x reference)
import functools

import jax
import jax.numpy as jnp
from jax import lax
from jax.experimental import pallas as pl
from jax.experimental.pallas import tpu as pltpu
from jax.experimental.pallas import tpu_sc as plsc

_F32 = jnp.float32
_U32 = jnp.uint32


def _proj_body(x_ref, wlo_ref, whi_ref, o_ref):
    g_lo = jnp.dot(x_ref[0], wlo_ref[...], preferred_element_type=_F32)
    g_hi = jnp.dot(x_ref[0], whi_ref[...], preferred_element_type=_F32)
    ul = lax.bitcast_convert_type(g_lo, _U32)
    uh = lax.bitcast_convert_type(g_hi, _U32)
    rl = ul + jnp.uint32(0x7FFF) + ((ul >> 16) & jnp.uint32(1))
    rh = uh + jnp.uint32(0x7FFF) + ((uh >> 16) & jnp.uint32(1))
    packed = ((rl >> 16) & jnp.uint32(0xFFFF)) | (rh & jnp.uint32(0xFFFF0000))
    o_ref[0] = lax.bitcast_convert_type(packed, _F32)


def _project_nodes_packed(node_feat, Wlo, Whi, block_n):
    B, N, D = node_feat.shape
    W = Wlo.shape[1]
    grid = (B, N // block_n)
    return pl.pallas_call(
        _proj_body,
        grid=grid,
        in_specs=[
            pl.BlockSpec((1, block_n, D), lambda b, n: (b, n, 0)),
            pl.BlockSpec((D, W), lambda b, n: (0, 0)),
            pl.BlockSpec((D, W), lambda b, n: (0, 0)),
        ],
        out_specs=pl.BlockSpec((1, block_n, W), lambda b, n: (b, n, 0)),
        out_shape=jax.ShapeDtypeStruct((B, N, W), _F32),
    )(node_feat, Wlo, Whi)


def _make_sc_gather_sum(B, N, T, W, C):
    info = plsc.get_sparse_core_info()
    NW = info.num_cores * info.num_subcores
    tpb = T // C
    n_chunks = B * tpb
    n_iters = (n_chunks + NW - 1) // NW
    assert n_iters >= 4 and (n_iters - 4) % 2 == 0
    mesh = plsc.VectorSubcoreMesh(core_axis_name="c", subcore_axis_name="s")

    @functools.partial(
        pl.kernel,
        mesh=mesh,
        compiler_params=pltpu.CompilerParams(use_tc_tiling_on_sc=False),
        out_type=jax.ShapeDtypeStruct((B, T, W), _F32),
        scratch_types=(
            [pltpu.VMEM((C,), jnp.int32) for _ in range(6)]
            + [pltpu.VMEM((C, W), _F32) for _ in range(6)]
            + [pltpu.SemaphoreType.DMA for _ in range(6)]
        ),
    )
    def sc_kernel(table_hbm, idx_hbm, out_hbm,
                  ia0, ia1, ia2, ib0, ib1, ib2,
                  ra0, ra1, ra2, rb0, rb1, rb2,
                  sia, sib, sga, sgb, swa, swb):
        wid = lax.axis_index("s") * info.num_cores + lax.axis_index("c")
        idx = ((ia0, ia1, ia2), (ib0, ib1, ib2))
        rows = ((ra0, ra1, ra2), (rb0, rb1, rb2))
        si = (sia, sib)
        sg = (sga, sgb)
        sw = (swa, swb)

        def coords(it):
            chunk = it * NW + wid
            chunk = jnp.where(chunk < n_chunks, chunk, wid)
            b = chunk // tpb
            t0 = (chunk % tpb) * C
            return b, t0

        def fire_idx(it, s):
            b, t0 = coords(it)
            ib = b * (3 * T) + t0
            for k in range(3):
                pltpu.async_copy(idx_hbm.at[pl.ds(ib + k * T, C)], idx[s][k],
                                 si[s])

        def wait_idx_adjust(it, s):
            for k in range(3):
                pltpu.make_async_copy(idx_hbm.at[pl.ds(k * C, C)], idx[s][k],
                                      si[s]).wait()
            b, _ = coords(it)
            base = b * (3 * N)

            def adj(j, _):
                sl = pl.ds(j * 16, 16)
                for k in range(3):
                    idx[s][k][sl] = idx[s][k][sl] * 3 + (base + k)
                return 0

            lax.fori_loop(0, C // 16, adj, 0)

        def fire_gathers(s):
            for k in range(3):
                pltpu.async_copy(table_hbm.at[idx[s][k]], rows[s][k], sg[s])

        def wait_gathers(s):
            for k in range(3):
                pltpu.make_async_copy(table_hbm.at[idx[s][k]], rows[s][k],
                                      sg[s]).wait()

        def add_rows(s):
            r0, r1, r2 = rows[s]
            c16 = jnp.uint32(16)
            cm = jnp.uint32(0xFFFF0000)
            cr = jnp.uint32(0x7FFF)
            c1 = jnp.uint32(1)

            def addrow(r, _):
                for cc in range(W // 16):
                    sl = pl.ds(cc * 16, 16)
                    u0 = lax.bitcast_convert_type(r0[r, sl], _U32)
                    u1 = lax.bitcast_convert_type(r1[r, sl], _U32)
                    u2 = lax.bitcast_convert_type(r2[r, sl], _U32)
                    lo = (lax.bitcast_convert_type(u0 << c16, _F32)
                          + lax.bitcast_convert_type(u1 << c16, _F32)
                          + lax.bitcast_convert_type(u2 << c16, _F32))
                    hi = (lax.bitcast_convert_type(u0 & cm, _F32)
                          + lax.bitcast_convert_type(u1 & cm, _F32)
                          + lax.bitcast_convert_type(u2 & cm, _F32))
                    ul = lax.bitcast_convert_type(lo, _U32)
                    uh = lax.bitcast_convert_type(hi, _U32)
                    rl = ul + cr + ((ul >> c16) & c1)
                    rh = uh + cr + ((uh >> c16) & c1)
                    r0[r, sl] = lax.bitcast_convert_type(
                        ((rl >> c16) & jnp.uint32(0xFFFF)) | (rh & cm), _F32)
                return 0

            lax.fori_loop(0, C, addrow, 0)

        def fire_wb(it, s):
            b, t0 = coords(it)
            pltpu.async_copy(rows[s][0], out_hbm.at[b, pl.ds(t0, C)], sw[s])

        def wait_wb(s):
            pltpu.make_async_copy(rows[s][0], out_hbm.at[0, pl.ds(0, C)],
                                  sw[s]).wait()

        fire_idx(0, 0)
        fire_idx(1, 1)
        wait_idx_adjust(0, 0)
        fire_gathers(0)

        def steady(it, s, first=False, fire_next=True, fire_idx_next=True):
            other = 1 - s
            if fire_next:
                wait_idx_adjust(it + 1, other)
                if not first:
                    wait_wb(other)
                fire_gathers(other)
            wait_gathers(s)
            if fire_next and fire_idx_next:
                fire_idx(it + 2, s)
            add_rows(s)
            fire_wb(it, s)

        steady(0, 0, first=True)
        steady(1, 1)

        def pair(j2, _):
            it = 2 + 2 * j2
            steady(it, 0)
            steady(it + 1, 1)
            return 0

        lax.fori_loop(0, (n_iters - 4) // 2, pair, 0)

        steady(n_iters - 2, 0, fire_idx_next=False)
        steady(n_iters - 1, 1, fire_next=False)
        wait_wb(0)
        wait_wb(1)

    return sc_kernel


def _head_body(h1_ref, b1lo_ref, b1hi_ref, A_ref, Bm_ref, b2_ref, Wc8_ref,
               bw_ref, bn_ref, wout_ref, nout_ref):
    u = lax.bitcast_convert_type(h1_ref[0], _U32)
    lo = lax.bitcast_convert_type(u << 16, _F32) + b1lo_ref[...]
    hi = lax.bitcast_convert_type(u & jnp.uint32(0xFFFF0000), _F32) + b1hi_ref[...]
    lo = lo * lax.logistic(lo)
    hi = hi * lax.logistic(hi)
    h = (jnp.dot(lo, A_ref[...], preferred_element_type=_F32)
         + jnp.dot(hi, Bm_ref[...], preferred_element_type=_F32)) + b2_ref[...]
    h = h * lax.logistic(h)
    o = lax.dot_general(Wc8_ref[...], h, (((1,), (1,)), ((), ())),
                        preferred_element_type=_F32)
    wrow = o[0:1, :] + bw_ref[0]
    nrows = o[1:4, :]
    ii = lax.broadcasted_iota(jnp.int32, nrows.shape, 0)
    bnv = jnp.where(ii == 0, bn_ref[0],
                    jnp.where(ii == 1, bn_ref[1], bn_ref[2]))
    nrows = nrows + bnv
    wout_ref[0] = lax.logistic(wrow)
    nv = jnp.sqrt(jnp.sum(nrows * nrows, axis=0, keepdims=True))
    ok = nv > 1e-8
    safe = jnp.where(ok, nv, 1.0)
    nout_ref[0] = jnp.where(ok, nrows / safe, 0.0)


def _head_packed(h1p, b1, W2, b2, Ww, bw, Wn, bn, block_t):
    B, T, Wd = h1p.shape
    H = W2.shape[0]
    grid = (B, T // block_t)
    Wc8 = jnp.zeros((8, H), _F32).at[0:1].set(Ww).at[1:4].set(Wn)
    A = W2[:, : Wd].T
    Bm = W2[:, Wd:].T

    def full(shape):
        return pl.BlockSpec(shape, lambda b, t: tuple(0 for _ in shape))

    wt, nt = pl.pallas_call(
        _head_body,
        grid=grid,
        in_specs=[
            pl.BlockSpec((1, block_t, Wd), lambda b, t: (b, t, 0)),
            full((1, Wd)),
            full((1, Wd)),
            full((Wd, H)),
            full((Wd, H)),
            full((1, H)),
            full((8, H)),
            pl.BlockSpec(memory_space=pltpu.SMEM),
            pl.BlockSpec(memory_space=pltpu.SMEM),
        ],
        out_specs=[
            pl.BlockSpec((1, 1, block_t), lambda b, t: (b, 0, t)),
            pl.BlockSpec((1, 3, block_t), lambda b, t: (b, 0, t)),
        ],
        out_shape=[
            jax.ShapeDtypeStruct((B, 1, T), _F32),
            jax.ShapeDtypeStruct((B, 3, T), _F32),
        ],
    )(h1p, b1[:Wd].reshape(1, Wd), b1[Wd:].reshape(1, Wd), A, Bm,
      b2.reshape(1, H), Wc8, bw, bn)
    return wt.transpose(0, 2, 1), nt.transpose(0, 2, 1)


def kernel(node_feat, tri_indices, W1, b1, W2, b2, Ww, bw, Wn, bn):
    B, N, D = node_feat.shape
    T = tri_indices.shape[1]
    H = W1.shape[0]

    Wcat = W1.reshape(H, 3, D).transpose(2, 1, 0).reshape(D, 3 * H)
    Wg = Wcat.reshape(D, 3, H)
    Wlo = Wg[:, :, : H // 2].reshape(D, 3 * (H // 2))
    Whi = Wg[:, :, H // 2:].reshape(D, 3 * (H // 2))
    idx_t = tri_indices.astype(jnp.int32).transpose(0, 2, 1).reshape(-1)

    G = _project_nodes_packed(node_feat, Wlo, Whi, block_n=2000)
    table = G.reshape(B * N * 3, H // 2)
    h1p = _make_sc_gather_sum(B, N, T, H // 2, C=128)(table, idx_t)
    weights, normals = _head_packed(h1p, b1, W2, b2, Ww, bw, Wn, bn,
                                    block_t=3200)
    return weights, normals


# speedup vs baseline: 1.1471x; 1.1471x over previous
import functools

import jax
import jax.numpy as jnp
from jax import lax
from jax.experimental import pallas as pl
from jax.experimental.pallas import tpu as pltpu
from jax.experimental.pallas import tpu_sc as plsc

_F32 = jnp.float32


def _proj_body(x_ref, w_ref, o_ref):
    o_ref[0] = jnp.dot(x_ref[0], w_ref[...], preferred_element_type=_F32)


def _project_nodes(node_feat, Wcat, block_n):
    B, N, D = node_feat.shape
    H3 = Wcat.shape[1]
    grid = (B, N // block_n)
    return pl.pallas_call(
        _proj_body,
        grid=grid,
        in_specs=[
            pl.BlockSpec((1, block_n, D), lambda b, n: (b, n, 0)),
            pl.BlockSpec((D, H3), lambda b, n: (0, 0)),
        ],
        out_specs=pl.BlockSpec((1, block_n, H3), lambda b, n: (b, n, 0)),
        out_shape=jax.ShapeDtypeStruct((B, N, H3), _F32),
    )(node_feat, Wcat)


def _make_sc_gather_sum(B, N, T, H, C, toff, tlen):
    info = plsc.get_sparse_core_info()
    NW = info.num_cores * info.num_subcores
    tpb = tlen // C
    n_chunks = B * tpb
    n_iters = (n_chunks + NW - 1) // NW
    assert n_iters >= 4 and n_chunks >= NW
    mesh = plsc.VectorSubcoreMesh(core_axis_name="c", subcore_axis_name="s")

    @functools.partial(
        pl.kernel,
        mesh=mesh,
        out_type=jax.ShapeDtypeStruct((B, tlen, H), _F32),
        scratch_types=(
            [pltpu.VMEM((C,), jnp.int32) for _ in range(6)]
            + [pltpu.VMEM((C, H), _F32) for _ in range(6)]
            + [pltpu.SemaphoreType.DMA for _ in range(6)]
        ),
    )
    def sc_kernel(table_hbm, idx_hbm, out_hbm,
                  ia0, ia1, ia2, ib0, ib1, ib2,
                  ra0, ra1, ra2, rb0, rb1, rb2,
                  sia, sib, sga, sgb, swa, swb):
        wid = lax.axis_index("s") * info.num_cores + lax.axis_index("c")
        idx = ((ia0, ia1, ia2), (ib0, ib1, ib2))
        rows = ((ra0, ra1, ra2), (rb0, rb1, rb2))
        si = (sia, sib)
        sg = (sga, sgb)
        sw = (swa, swb)

        def coords(it):
            chunk = it * NW + wid
            chunk = jnp.where(chunk < n_chunks, chunk, wid)
            b = chunk // tpb
            t0 = (chunk % tpb) * C
            return b, t0

        def fire_idx(it, s):
            b, t0 = coords(it)
            ib = b * (3 * T) + toff + t0
            for k in range(3):
                pltpu.async_copy(idx_hbm.at[pl.ds(ib + k * T, C)], idx[s][k],
                                 si[s])

        def wait_idx_adjust(it, s):
            for k in range(3):
                pltpu.make_async_copy(idx_hbm.at[pl.ds(k * C, C)], idx[s][k],
                                      si[s]).wait()
            b, _ = coords(it)
            base = b * (3 * N)

            def adj(j, _):
                sl = pl.ds(j * 16, 16)
                for k in range(3):
                    idx[s][k][sl] = idx[s][k][sl] * 3 + (base + k)
                return 0

            lax.fori_loop(0, C // 16, adj, 0)

        def fire_gathers(s):
            for k in range(3):
                pltpu.async_copy(table_hbm.at[idx[s][k]], rows[s][k], sg[s])

        def wait_gathers(s):
            for k in range(3):
                pltpu.make_async_copy(table_hbm.at[idx[s][k]], rows[s][k],
                                      sg[s]).wait()

        def add_rows(s):
            r0, r1, r2 = rows[s]

            def addrow(r, _):
                for cc in range(H // 16):
                    sl = pl.ds(cc * 16, 16)
                    plsc.addupdate(r0.at[r, sl], r1[r, sl])
                    plsc.addupdate(r0.at[r, sl], r2[r, sl])
                return 0

            lax.fori_loop(0, C, addrow, 0)

        def fire_wb(it, s):
            b, t0 = coords(it)
            pltpu.async_copy(rows[s][0], out_hbm.at[b, pl.ds(t0, C)], sw[s])

        def wait_wb(s):
            pltpu.make_async_copy(rows[s][0], out_hbm.at[0, pl.ds(0, C)],
                                  sw[s]).wait()

        fire_idx(0, 0)
        fire_idx(1, 1)
        wait_idx_adjust(0, 0)
        fire_gathers(0)

        def steady(it, s, first=False, fire_next=True, fire_idx_next=True):
            other = 1 - s
            if fire_next:
                wait_idx_adjust(it + 1, other)
                if not first:
                    wait_wb(other)
                fire_gathers(other)
            wait_gathers(s)
            if fire_next and fire_idx_next:
                fire_idx(it + 2, s)
            add_rows(s)
            fire_wb(it, s)

        steady(0, 0, first=True)
        steady(1, 1)

        n_pairs = (n_iters - 4) // 2

        def pair(j2, _):
            it = 2 + 2 * j2
            steady(it, 0)
            steady(it + 1, 1)
            return 0

        if n_pairs > 0:
            lax.fori_loop(0, n_pairs, pair, 0)

        for it in range(2 + 2 * n_pairs, n_iters):
            steady(it, it % 2, fire_next=(it < n_iters - 1),
                   fire_idx_next=(it + 2 < n_iters))
        wait_wb(0)
        wait_wb(1)

    return sc_kernel


def _head_body(h1_ref, b1_ref, W2_ref, b2_ref, Wc8_ref, bw_ref, bn_ref,
               wout_ref, nout_ref):
    h = h1_ref[0] + b1_ref[...]
    h = h * lax.logistic(h)
    h = lax.dot_general(h, W2_ref[...], (((1,), (1,)), ((), ())),
                        preferred_element_type=_F32) + b2_ref[...]
    h = h * lax.logistic(h)
    o = lax.dot_general(Wc8_ref[...], h, (((1,), (1,)), ((), ())),
                        preferred_element_type=_F32)
    wrow = o[0:1, :] + bw_ref[0]
    nrows = o[1:4, :]
    ii = lax.broadcasted_iota(jnp.int32, nrows.shape, 0)
    bnv = jnp.where(ii == 0, bn_ref[0],
                    jnp.where(ii == 1, bn_ref[1], bn_ref[2]))
    nrows = nrows + bnv
    wout_ref[0] = lax.logistic(wrow)
    nv = jnp.sqrt(jnp.sum(nrows * nrows, axis=0, keepdims=True))
    ok = nv > 1e-8
    safe = jnp.where(ok, nv, 1.0)
    nout_ref[0] = jnp.where(ok, nrows / safe, 0.0)


def _head(h1, b1, W2, b2, Wc8, bw, bn, block_t):
    B, T, H = h1.shape
    grid = (B, T // block_t)

    def full(shape):
        return pl.BlockSpec(shape, lambda b, t: tuple(0 for _ in shape))

    return pl.pallas_call(
        _head_body,
        grid=grid,
        in_specs=[
            pl.BlockSpec((1, block_t, H), lambda b, t: (b, t, 0)),
            full((1, H)),
            full((H, H)),
            full((1, H)),
            full((8, H)),
            pl.BlockSpec(memory_space=pltpu.SMEM),
            pl.BlockSpec(memory_space=pltpu.SMEM),
        ],
        out_specs=[
            pl.BlockSpec((1, 1, block_t), lambda b, t: (b, 0, t)),
            pl.BlockSpec((1, 3, block_t), lambda b, t: (b, 0, t)),
        ],
        out_shape=[
            jax.ShapeDtypeStruct((B, 1, T), _F32),
            jax.ShapeDtypeStruct((B, 3, T), _F32),
        ],
    )(h1, b1.reshape(1, H), W2, b2.reshape(1, H), Wc8, bw, bn)


def kernel(node_feat, tri_indices, W1, b1, W2, b2, Ww, bw, Wn, bn):
    B, N, D = node_feat.shape
    T = tri_indices.shape[1]
    H = W1.shape[0]

    Wcat = W1.reshape(H, 3, D).transpose(2, 1, 0).reshape(D, 3 * H)
    idx_t = tri_indices.astype(jnp.int32).transpose(0, 2, 1).reshape(-1)
    Wc8 = jnp.zeros((8, H), _F32).at[0:1].set(Ww).at[1:4].set(Wn)

    G = _project_nodes(node_feat, Wcat, block_n=2000)
    table = G.reshape(B * N * 3, H)

    segs = ((0, 40960, 2048), (40960, 39040, 7808)) if T == 80000 else \
        ((0, T, T),)
    h1s = [
        _make_sc_gather_sum(B, N, T, H, 128, toff, tlen)(table, idx_t)
        for toff, tlen, _ in segs
    ]
    wts, nts = [], []
    for (toff, tlen, blk), h1 in zip(segs, h1s):
        wt, nt = _head(h1, b1, W2, b2, Wc8, bw, bn, block_t=blk)
        wts.append(wt)
        nts.append(nt)
    weights = jnp.concatenate(wts, axis=2).transpose(0, 2, 1)
    normals = jnp.concatenate(nts, axis=2).transpose(0, 2, 1)
    return weights, normals
